# baseline (speedup 1.0000x reference)
import functools

import jax
import jax.numpy as jnp
from jax.experimental import pallas as pl
from jax.experimental.pallas import tpu as pltpu

_N = 4096
_D = 128
_ALPHA = 3.0
_K = 64
_R = 256


@functools.cache
def _dope_scaled():
    dope = jax.random.uniform(jax.random.key(42), (_N, _N), dtype=jnp.float32)
    return dope * 0.0001


def _stage1_body(e1_ref, e2_ref, w1_ref, b1_ref, w2_ref, b2_ref, m1_ref, m2_ref):
    dn = (((1,), (1,)), ((), ()))
    x1 = jax.lax.dot_general(e1_ref[...], w1_ref[...], dn,
                             preferred_element_type=jnp.float32)
    x2 = jax.lax.dot_general(e2_ref[...], w2_ref[...], dn,
                             preferred_element_type=jnp.float32)
    m1_ref[...] = jnp.tanh(_ALPHA * (x1 + b1_ref[...]))
    m2_ref[...] = jnp.tanh(_ALPHA * (x2 + b2_ref[...]))


def _stage2_body(m1_ref, m2_ref, dope_ref, out_ref):
    i = pl.program_id(0)
    m1 = m1_ref[...]
    m2 = m2_ref[...]
    m1_blk = m1_ref[pl.ds(i * _R, _R), :]
    m2_blk = m2_ref[pl.ds(i * _R, _R), :]
    dn = (((1,), (1,)), ((), ()))
    x = jax.lax.dot_general(m1_blk, m2, dn, preferred_element_type=jnp.float32)
    y = jax.lax.dot_general(m2_blk, m1, dn, preferred_element_type=jnp.float32)
    a_soft = jax.nn.relu(jnp.tanh(_ALPHA * (x - y)))
    a_doped = a_soft + dope_ref[...]
    bits = jax.lax.bitcast_convert_type(a_doped, jnp.int32)

    ones_col = jnp.ones((_N, 1), jnp.float32)

    def row_count(pred):
        return jax.lax.dot_general(pred.astype(jnp.float32), ones_col,
                                   (((1,), (0,)), ((), ())),
                                   preferred_element_type=jnp.float32)

    a3 = a_doped.reshape(_R, 32, 128)
    cmax = jnp.max(a3, axis=2)
    rmax = jnp.max(cmax, axis=1, keepdims=True)
    m2c = jnp.max(jnp.where(a3 < cmax[:, :, None], a3, 0.0), axis=2)
    lbf = jnp.min(m2c, axis=1, keepdims=True)
    lo0 = jax.lax.bitcast_convert_type(lbf, jnp.int32)
    hi0 = jax.lax.bitcast_convert_type(rmax, jnp.int32) + 1

    kf = jnp.float32(_K)

    def cond(c):
        lo, hi = c
        return jnp.any(lo + 1 < hi)

    def body(c):
        lo, hi = c
        mid = lo + ((hi - lo) >> 1)
        take = row_count(bits >= mid) >= kf
        return (jnp.where(take, mid, lo), jnp.where(take, hi, mid))

    lo, _ = jax.lax.while_loop(cond, body, (lo0, hi0))
    t = lo
    gt = bits > t
    c1 = row_count(gt)
    quota = kf - c1
    eq = bits == t

    eqf = eq.astype(jnp.float32)
    eq3 = eqf.reshape(_R, 32, 128)
    csum = jnp.sum(eq3, axis=2)
    ccum = csum
    for s in (1, 2, 4, 8, 16):
        ccum = ccum + jnp.concatenate(
            [jnp.zeros((_R, s), jnp.float32), ccum[:, :-s]], axis=1)
    chstar = jnp.sum((ccum < quota).astype(jnp.int32), axis=1, keepdims=True)
    ccum_excl = ccum - csum
    chiota = jax.lax.broadcasted_iota(jnp.int32, (_R, 32), 1)
    onehot = (chiota == chstar).astype(jnp.float32)
    base_cnt = jnp.sum(ccum_excl * onehot, axis=1, keepdims=True)
    chunk_eq = jnp.sum(eq3 * onehot[:, :, None], axis=1)
    quota2 = quota - base_cnt

    col128 = jax.lax.broadcasted_iota(jnp.int32, (_R, 128), 1)

    def ibody(_, c):
        clo, chi = c
        cmid = clo + ((chi - clo) >> 1)
        g = jnp.sum(jnp.where(col128 < cmid, chunk_eq, 0.0), axis=1,
                    keepdims=True)
        enough = g >= quota2
        return (jnp.where(enough, clo, cmid), jnp.where(enough, cmid, chi))

    clo0 = jnp.zeros((_R, 1), jnp.int32)
    chi0 = jnp.full((_R, 1), 128, jnp.int32)
    _, c2 = jax.lax.fori_loop(0, 7, ibody, (clo0, chi0))
    cstar = chstar * 128 + c2

    col = jax.lax.broadcasted_iota(jnp.int32, (_R, _N), 1)
    mask = gt | (eq & (col < cstar))
    out_ref[...] = jnp.where(mask, a_soft, 0.0)


def kernel(node_idx, src_emb, tgt_emb, src_W, src_b, tgt_W, tgt_b):
    e1 = jnp.take(src_emb, node_idx, axis=0)
    e2 = jnp.take(tgt_emb, node_idx, axis=0)
    b1 = src_b.reshape(1, _D)
    b2 = tgt_b.reshape(1, _D)

    m1, m2 = pl.pallas_call(
        _stage1_body,
        out_shape=[
            jax.ShapeDtypeStruct((_N, _D), jnp.float32),
            jax.ShapeDtypeStruct((_N, _D), jnp.float32),
        ],
    )(e1, e2, src_W, b1, tgt_W, b2)

    grid = (_N // _R,)
    a = pl.pallas_call(
        _stage2_body,
        grid=grid,
        in_specs=[
            pl.BlockSpec((_N, _D), lambda i: (0, 0)),
            pl.BlockSpec((_N, _D), lambda i: (0, 0)),
            pl.BlockSpec((_R, _N), lambda i: (i, 0)),
        ],
        out_specs=pl.BlockSpec((_R, _N), lambda i: (i, 0)),
        out_shape=jax.ShapeDtypeStruct((_N, _N), jnp.float32),
    )(m1, m2, _dope_scaled())
    return a


# speedup vs baseline: 1.5263x; 1.5263x over previous
import functools

import jax
import jax.numpy as jnp
from jax.experimental import pallas as pl
from jax.experimental.pallas import tpu as pltpu

_N = 4096
_D = 128
_ALPHA = 3.0
_K = 64
_R = 256


@functools.cache
def _dope_scaled():
    dope = jax.random.uniform(jax.random.key(42), (_N, _N), dtype=jnp.float32)
    return dope * 0.0001


def _stage1_body(e1_ref, e2_ref, w1_ref, b1_ref, w2_ref, b2_ref, m1_ref, m2_ref):
    dn = (((1,), (1,)), ((), ()))
    x1 = jax.lax.dot_general(e1_ref[...], w1_ref[...], dn,
                             preferred_element_type=jnp.float32)
    x2 = jax.lax.dot_general(e2_ref[...], w2_ref[...], dn,
                             preferred_element_type=jnp.float32)
    m1_ref[...] = jnp.tanh(_ALPHA * (x1 + b1_ref[...]))
    m2_ref[...] = jnp.tanh(_ALPHA * (x2 + b2_ref[...]))


def _stage2_body(m1_ref, m2_ref, dope_ref, out_ref):
    i = pl.program_id(0)
    m1 = m1_ref[...]
    m2 = m2_ref[...]
    m1_blk = m1_ref[pl.ds(i * _R, _R), :]
    m2_blk = m2_ref[pl.ds(i * _R, _R), :]
    dn = (((1,), (1,)), ((), ()))
    x = jax.lax.dot_general(m1_blk, m2, dn, preferred_element_type=jnp.float32)
    y = jax.lax.dot_general(m2_blk, m1, dn, preferred_element_type=jnp.float32)
    a_soft = jax.nn.relu(jnp.tanh(_ALPHA * (x - y)))
    a_doped = a_soft + dope_ref[...]
    bits = jax.lax.bitcast_convert_type(a_doped, jnp.int32)

    a3 = a_doped.reshape(_R, 32, 128)
    cmax = jnp.max(a3, axis=2)
    rmax = jnp.max(cmax, axis=1, keepdims=True)
    m2c = jnp.max(jnp.where(a3 < cmax[:, :, None], a3, 0.0), axis=2)
    lbf = jnp.min(m2c, axis=1, keepdims=True)
    lo0 = jax.lax.bitcast_convert_type(lbf, jnp.int32)
    hi0 = jax.lax.bitcast_convert_type(rmax, jnp.int32) + 1

    def cond(c):
        lo, hi = c
        return jnp.any(lo + 1 < hi)

    def body(c):
        lo, hi = c
        mid = lo + ((hi - lo) >> 1)
        cnt = jnp.sum((bits >= mid).astype(jnp.int32), axis=1, keepdims=True)
        take = cnt >= _K
        return (jnp.where(take, mid, lo), jnp.where(take, hi, mid))

    lo, _ = jax.lax.while_loop(cond, body, (lo0, hi0))
    t = lo
    gt = bits > t
    c1 = jnp.sum(gt.astype(jnp.int32), axis=1, keepdims=True)
    quota = (_K - c1).astype(jnp.float32)
    eq = bits == t

    eqf = eq.astype(jnp.float32)
    eq3 = eqf.reshape(_R, 32, 128)
    tri128 = (jax.lax.broadcasted_iota(jnp.int32, (128, 128), 0)
              <= jax.lax.broadcasted_iota(jnp.int32, (128, 128), 1)
              ).astype(jnp.float32)
    pre3 = jax.lax.dot_general(eq3, tri128, (((2,), (0,)), ((), ())),
                               preferred_element_type=jnp.float32)
    csum = pre3[:, :, 127]
    tri32 = (jax.lax.broadcasted_iota(jnp.int32, (32, 32), 0)
             <= jax.lax.broadcasted_iota(jnp.int32, (32, 32), 1)
             ).astype(jnp.float32)
    ccum = jax.lax.dot_general(csum, tri32, (((1,), (0,)), ((), ())),
                               preferred_element_type=jnp.float32)
    excl = ccum - csum
    prefix = (pre3 + excl[:, :, None]).reshape(_R, _N)
    mask = gt | (eq & (prefix <= quota))
    out_ref[...] = jnp.where(mask, a_soft, 0.0)


def kernel(node_idx, src_emb, tgt_emb, src_W, src_b, tgt_W, tgt_b):
    e1 = jnp.take(src_emb, node_idx, axis=0)
    e2 = jnp.take(tgt_emb, node_idx, axis=0)
    b1 = src_b.reshape(1, _D)
    b2 = tgt_b.reshape(1, _D)

    m1, m2 = pl.pallas_call(
        _stage1_body,
        out_shape=[
            jax.ShapeDtypeStruct((_N, _D), jnp.float32),
            jax.ShapeDtypeStruct((_N, _D), jnp.float32),
        ],
    )(e1, e2, src_W, b1, tgt_W, b2)

    grid = (_N // _R,)
    a = pl.pallas_call(
        _stage2_body,
        grid=grid,
        in_specs=[
            pl.BlockSpec((_N, _D), lambda i: (0, 0)),
            pl.BlockSpec((_N, _D), lambda i: (0, 0)),
            pl.BlockSpec((_R, _N), lambda i: (i, 0)),
        ],
        out_specs=pl.BlockSpec((_R, _N), lambda i: (i, 0)),
        out_shape=jax.ShapeDtypeStruct((_N, _N), jnp.float32),
    )(m1, m2, _dope_scaled())
    return a
